# Initial kernel scaffold; baseline (speedup 1.0000x reference)
import jax
import jax.numpy as jnp
from jax.experimental import pallas as pl


def kernel(z, edge_label_index, rel_emb_weight):
    raise NotImplementedError("write your pallas kernel here")


# speedup vs baseline: 1.1607x; 1.1607x over previous
import functools

import jax
import jax.numpy as jnp
from jax import lax
from jax.experimental import pallas as pl
from jax.experimental.pallas import tpu as pltpu
from jax.experimental.pallas import tpu_sc as plsc

_NC = 2
_NS = 16
_NW = _NC * _NS
_C = 80
_L = 16


def _scale_body(z_ref, r_ref, o_ref):
    o_ref[...] = z_ref[...] * r_ref[...]


def _prescale(z, rel_emb_weight):
    return pl.pallas_call(
        _scale_body,
        out_shape=jax.ShapeDtypeStruct(z.shape, z.dtype),
    )(z, rel_emb_weight)


def _make_sc_kernel(n_chunk_rows, d):
    ch_per_w = n_chunk_rows // _NW
    mesh = plsc.VectorSubcoreMesh(core_axis_name="c", subcore_axis_name="s")

    @functools.partial(
        pl.kernel,
        mesh=mesh,
        compiler_params=pltpu.CompilerParams(needs_layout_passes=False),
        out_type=jax.ShapeDtypeStruct((_NW, ch_per_w, _C), jnp.float32),
        scratch_types=[
            pltpu.VMEM((ch_per_w, _C), jnp.int32),
            pltpu.VMEM((ch_per_w, _C), jnp.int32),
            pltpu.VMEM((_C, d), jnp.float32),
            pltpu.VMEM((_C, d), jnp.float32),
            pltpu.VMEM((ch_per_w, _C), jnp.float32),
            pltpu.SemaphoreType.DMA,
        ],
    )
    def sck(zr_hbm, z_hbm, h_hbm, t_hbm, out_hbm,
            ih_v, it_v, hr_v, tr_v, out_v, sem):
        wid = lax.axis_index("s") * _NC + lax.axis_index("c")
        pltpu.sync_copy(h_hbm.at[wid], ih_v)
        pltpu.sync_copy(t_hbm.at[wid], it_v)
        iota = lax.iota(jnp.int32, _L)

        def chunk(k, carry):
            cp_h = pltpu.async_copy(zr_hbm.at[ih_v.at[k]], hr_v, sem)
            cp_t = pltpu.async_copy(z_hbm.at[it_v.at[k]], tr_v, sem)
            cp_h.wait()
            cp_t.wait()
            for g in range(_C // _L):
                rows = iota + (g * _L)
                acc = jnp.zeros((_L,), jnp.float32)
                for dd in range(d):
                    cols = jnp.full((_L,), dd, jnp.int32)
                    hv = plsc.load_gather(hr_v, [rows, cols])
                    tv = plsc.load_gather(tr_v, [rows, cols])
                    acc = acc + hv * tv
                out_v[k, pl.ds(g * _L, _L)] = acc
            return carry

        lax.fori_loop(0, ch_per_w, chunk, 0)
        pltpu.sync_copy(out_v, out_hbm.at[wid])

    return sck


def kernel(z, edge_label_index, rel_emb_weight):
    n_nodes, d = z.shape
    n_edges = edge_label_index.shape[1]
    zr = _prescale(z, rel_emb_weight)
    heads = edge_label_index[0].reshape(_NW, -1, _C)
    tails = edge_label_index[1].reshape(_NW, -1, _C)
    sck = _make_sc_kernel(_NW * heads.shape[1], d)
    out2 = sck(zr, z, heads, tails)
    return out2.reshape(n_edges)
